# Initial kernel scaffold; baseline (speedup 1.0000x reference)
import jax
import jax.numpy as jnp
from jax.experimental import pallas as pl


def kernel(x, edge_index, W, b):
    raise NotImplementedError("write your pallas kernel here")


# speedup vs baseline: 19.4374x; 19.4374x over previous
import functools

import jax
import jax.numpy as jnp
from jax import lax
from jax.experimental import pallas as pl
from jax.experimental.pallas import tpu as pltpu
from jax.experimental.pallas import tpu_sc as plsc

_NC = 2
_NS = 16
_LANES = 16
_CH = 128


def _partition(e, n_nodes):
    n_chunks = e // _CH
    assert n_chunks * _CH == e
    per_sc = n_chunks // _NC
    assert per_sc * _NC == n_chunks
    per_tile = per_sc // _NS
    rem = per_sc - per_tile * _NS
    rows_main = (n_nodes // _NS) // 8 * 8
    rows_tail = n_nodes - rows_main * _NS
    assert rows_tail % 8 == 0
    return per_sc, per_tile, rem, rows_main, rows_tail


def _sc_degree(dst, n_nodes):
    e = dst.shape[0]
    per_sc, per_tile, rem, rows_main, rows_tail = _partition(e, n_nodes)
    zn = rows_main + rows_tail

    mesh = plsc.VectorSubcoreMesh(core_axis_name="c", subcore_axis_name="s")

    @functools.partial(
        pl.kernel,
        out_type=(jax.ShapeDtypeStruct((n_nodes,), jnp.float32),
                  jax.ShapeDtypeStruct((n_nodes,), jnp.float32)),
        mesh=mesh,
        scratch_types=[
            pltpu.VMEM((1, _CH), jnp.int32),
            pltpu.VMEM((_CH,), jnp.float32),
            pltpu.VMEM((zn,), jnp.float32),
            pltpu.VMEM_SHARED((n_nodes,), jnp.float32),
        ],
    )
    def k(dst_hbm, out0_hbm, out1_hbm, idx_v, ones_v, zero_v, acc_sh):
        cid = lax.axis_index("c")
        sid = lax.axis_index("s")

        one16 = jnp.full((_LANES,), 1.0, jnp.float32)
        zero16 = jnp.zeros((_LANES,), jnp.float32)

        @pl.loop(0, _CH // _LANES)
        def _(i):
            ones_v[pl.ds(i * _LANES, _LANES)] = one16

        @pl.loop(0, zn // _LANES)
        def _(i):
            zero_v[pl.ds(i * _LANES, _LANES)] = zero16

        row0 = sid * rows_main
        pltpu.sync_copy(zero_v.at[pl.ds(0, rows_main)],
                        acc_sh.at[pl.ds(row0, rows_main)])

        @pl.when(sid == _NS - 1)
        def _():
            pltpu.sync_copy(zero_v.at[pl.ds(0, rows_tail)],
                            acc_sh.at[pl.ds(_NS * rows_main, rows_tail)])

        plsc.subcore_barrier()

        def chunk(c):
            base = c * _CH
            pltpu.sync_copy(dst_hbm.at[pl.ds(base, _CH)], idx_v.at[0])
            pltpu.sync_copy(ones_v, acc_sh.at[idx_v.at[0]], add=True)

        c0 = cid * per_sc + sid * per_tile

        @pl.loop(0, per_tile)
        def _(t):
            chunk(c0 + t)

        @pl.when(sid < rem)
        def _():
            chunk(cid * per_sc + _NS * per_tile + sid)

        plsc.subcore_barrier()

        def copy_out(out_hbm):
            pltpu.sync_copy(acc_sh.at[pl.ds(row0, rows_main)],
                            zero_v.at[pl.ds(0, rows_main)])
            pltpu.sync_copy(zero_v.at[pl.ds(0, rows_main)],
                            out_hbm.at[pl.ds(row0, rows_main)])

            @pl.when(sid == _NS - 1)
            def _():
                pltpu.sync_copy(
                    acc_sh.at[pl.ds(_NS * rows_main, rows_tail)],
                    zero_v.at[pl.ds(rows_main, rows_tail)],
                )
                pltpu.sync_copy(
                    zero_v.at[pl.ds(rows_main, rows_tail)],
                    out_hbm.at[pl.ds(_NS * rows_main, rows_tail)],
                )

        @pl.when(cid == 0)
        def _():
            copy_out(out0_hbm)

        @pl.when(cid == 1)
        def _():
            copy_out(out1_hbm)

    return k(dst)


def _sc_aggregate(g, src, dst, n_nodes):
    e = src.shape[0]
    d = g.shape[1]
    per_sc, per_tile, rem, rows_main, rows_tail = _partition(e, n_nodes)
    zb = 208
    nz = rows_main // zb
    assert zb * nz == rows_main and rows_tail <= zb

    mesh = plsc.VectorSubcoreMesh(core_axis_name="c", subcore_axis_name="s")

    @functools.partial(
        pl.kernel,
        out_type=jax.ShapeDtypeStruct((_NC, n_nodes, d), jnp.float32),
        mesh=mesh,
        scratch_types=[
            pltpu.VMEM((1, _CH), jnp.int32),
            pltpu.VMEM((1, _CH), jnp.int32),
            pltpu.VMEM((_CH, d), jnp.float32),
            pltpu.VMEM((zb, d), jnp.float32),
            pltpu.VMEM_SHARED((n_nodes, d), jnp.float32),
            pltpu.SemaphoreType.DMA,
        ],
    )
    def k(g_hbm, src_hbm, dst_hbm, out_hbm, sidx_v, didx_v, rows_v, zero_v,
          acc_sh, sem):
        cid = lax.axis_index("c")
        sid = lax.axis_index("s")

        zero16 = jnp.zeros((_LANES,), jnp.float32)

        @pl.loop(0, zb)
        def _(i):
            @pl.loop(0, d, step=_LANES)
            def _(j):
                zero_v[i, pl.ds(j, _LANES)] = zero16

        row0 = sid * rows_main

        @pl.loop(0, nz)
        def _(j):
            pltpu.sync_copy(zero_v, acc_sh.at[pl.ds(row0 + j * zb, zb)])

        @pl.when(sid == _NS - 1)
        def _():
            pltpu.sync_copy(zero_v.at[pl.ds(0, rows_tail)],
                            acc_sh.at[pl.ds(_NS * rows_main, rows_tail)])

        plsc.subcore_barrier()

        def chunk(c):
            base = c * _CH
            pltpu.sync_copy(src_hbm.at[pl.ds(base, _CH)], sidx_v.at[0])
            pltpu.sync_copy(dst_hbm.at[pl.ds(base, _CH)], didx_v.at[0])
            pltpu.async_copy(g_hbm.at[sidx_v.at[0]], rows_v, sem).wait()
            pltpu.sync_copy(rows_v, acc_sh.at[didx_v.at[0]], add=True)

        c0 = cid * per_sc + sid * per_tile

        @pl.loop(0, per_tile)
        def _(t):
            chunk(c0 + t)

        @pl.when(sid < rem)
        def _():
            chunk(cid * per_sc + _NS * per_tile + sid)

        plsc.subcore_barrier()
        pltpu.sync_copy(
            acc_sh.at[pl.ds(row0, rows_main)],
            out_hbm.at[cid].at[pl.ds(row0, rows_main)],
        )

        @pl.when(sid == _NS - 1)
        def _():
            pltpu.sync_copy(
                acc_sh.at[pl.ds(_NS * rows_main, rows_tail)],
                out_hbm.at[cid].at[pl.ds(_NS * rows_main, rows_tail)],
            )

    return k(g, src, dst)


def _dinv_from_parts(d0_ref, d1_ref):
    deg = d0_ref[...] + d1_ref[...]
    return jnp.where(deg > 0, lax.rsqrt(jnp.maximum(deg, 1e-12)), 0.0)


def _tc_prep(x, w, d0c, d1c):
    n, d = x.shape
    br = 2000
    assert n % br == 0

    def body(x_ref, w_ref, d0_ref, d1_ref, g_ref):
        dinv = _dinv_from_parts(d0_ref, d1_ref)
        h = jnp.dot(x_ref[...], w_ref[...], preferred_element_type=jnp.float32)
        g_ref[...] = h * dinv

    return pl.pallas_call(
        body,
        grid=(n // br,),
        in_specs=[
            pl.BlockSpec((br, d), lambda i: (i, 0)),
            pl.BlockSpec((d, d), lambda i: (0, 0)),
            pl.BlockSpec((br, 1), lambda i: (i, 0)),
            pl.BlockSpec((br, 1), lambda i: (i, 0)),
        ],
        out_specs=pl.BlockSpec((br, d), lambda i: (i, 0)),
        out_shape=jax.ShapeDtypeStruct((n, d), jnp.float32),
    )(x, w, d0c, d1c)


def _tc_finish(accp, d0c, d1c, b2):
    n, d = accp.shape[1], accp.shape[2]
    br = 2000
    assert n % br == 0

    def body(a_ref, d0_ref, d1_ref, b_ref, o_ref):
        dinv = _dinv_from_parts(d0_ref, d1_ref)
        o_ref[...] = (a_ref[0] + a_ref[1]) * dinv + b_ref[...]

    return pl.pallas_call(
        body,
        grid=(n // br,),
        in_specs=[
            pl.BlockSpec((_NC, br, d), lambda i: (0, i, 0)),
            pl.BlockSpec((br, 1), lambda i: (i, 0)),
            pl.BlockSpec((br, 1), lambda i: (i, 0)),
            pl.BlockSpec((1, d), lambda i: (0, 0)),
        ],
        out_specs=pl.BlockSpec((br, d), lambda i: (i, 0)),
        out_shape=jax.ShapeDtypeStruct((n, d), jnp.float32),
    )(accp, d0c, d1c, b2)


def kernel(x, edge_index, W, b):
    n, d = x.shape
    ei = edge_index.astype(jnp.int32)
    src = ei[0]
    dst = ei[1]
    deg0, deg1 = _sc_degree(dst, n)
    d0c = deg0.reshape(n, 1)
    d1c = deg1.reshape(n, 1)
    g = _tc_prep(x, W, d0c, d1c)
    accp = _sc_aggregate(g, src, dst, n)
    return _tc_finish(accp, d0c, d1c, b.reshape(1, d))
